# baseline (speedup 1.0000x reference)
import functools

import jax
import jax.numpy as jnp
import numpy as np
from jax import lax
from jax.experimental import pallas as pl
from jax.experimental.pallas import tpu as pltpu
from jax.experimental.pallas import tpu_sc as plsc

_BATCH = 65536
_N_PRED = 256
_COLS = (0, 3, 17, 42, 97, 128, 200, 255)
_SIGNS = (-1.0, 1.0, -1.0, 1.0, 1.0, -1.0, 1.0, -1.0)
_L = len(_COLS)
_MIN_W = 0.0
_MAX_W = 500.0

_IDX_CONST = np.asarray(_COLS, dtype=np.int32).reshape(-1, 1)

_BSC = 16384
_LANES = 16
_NUM_CORES = 2
_NUM_SUBCORES = 16
_NW = _NUM_CORES * _NUM_SUBCORES
_RPW = _BSC // _NW
_WPW = _RPW * _L
_CHUNK = 128
_CHUNK_W = _CHUNK * _N_PRED
_NCHUNK = _RPW // _CHUNK
_GROUPS = _CHUNK // _LANES

_COLTERM = tuple(((c >> 7) * 1024 + (c & 127)) for c in _COLS)


def _tec_body(ga_hbm, w_hbm, out_hbm, sa, sb, outv, wv, sema, semb):
    wid = lax.axis_index("s") * _NUM_CORES + lax.axis_index("c")
    wbase = wid * (_RPW * _N_PRED)

    pltpu.sync_copy(w_hbm, wv)
    w16 = wv[...]
    w16 = jnp.minimum(jnp.maximum(w16, _MIN_W), _MAX_W)

    lane = lax.broadcasted_iota(jnp.int32, (_LANES,), 0)
    laneoff = (lane >> 3) * 2048 + (lane & 7) * 128

    pltpu.make_async_copy(
        ga_hbm.at[pl.ds(wbase, _CHUNK_W)], sa, sema).start()
    pltpu.make_async_copy(
        ga_hbm.at[pl.ds(wbase + _CHUNK_W, _CHUNK_W)], sb, semb).start()

    def body(g, carry):
        for b, (buf, sem) in enumerate(((sa, sema), (sb, semb))):
            c = 2 * g + b
            slab0 = wbase + c * _CHUNK_W
            pltpu.make_async_copy(
                ga_hbm.at[pl.ds(slab0, _CHUNK_W)], buf, sem).wait()

            for s in range(_GROUPS):
                ivec = laneoff + (s * 4096)
                xs = []
                for j, sg in enumerate(_SIGNS):
                    x = plsc.load_gather(buf, [ivec + _COLTERM[j]])
                    xs.append(-x if sg < 0 else x)
                m = xs[0]
                for x in xs[1:]:
                    m = jnp.maximum(m, x)
                es = [jnp.exp(x - m) for x in xs]
                tot = es[0]
                for e in es[1:]:
                    tot = tot + e
                scale = w16 / tot
                off = c * 1024 + s * _LANES
                for j, sg in enumerate(_SIGNS):
                    d = es[j] * scale
                    if sg < 0:
                        d = -d
                    outv[pl.ds(off + j * 128, _LANES)] = d

            @pl.when(g < _NCHUNK // 2 - 1)
            def _prefetch():
                pltpu.make_async_copy(
                    ga_hbm.at[pl.ds(slab0 + 2 * _CHUNK_W, _CHUNK_W)], buf,
                    sem).start()
        return carry

    lax.fori_loop(0, _NCHUNK // 2, body, 0)
    pltpu.sync_copy(outv, out_hbm.at[pl.ds(wid * _WPW, _WPW)])


def _delta_sc(ga_lin, wvec):
    mesh = plsc.VectorSubcoreMesh(core_axis_name="c", subcore_axis_name="s")
    k = functools.partial(
        pl.kernel,
        mesh=mesh,
        compiler_params=pltpu.CompilerParams(
            use_tc_tiling_on_sc=False, needs_layout_passes=False),
        out_type=jax.ShapeDtypeStruct((_BSC * _L,), jnp.float32),
        scratch_types=[
            pltpu.VMEM((_CHUNK_W,), jnp.float32),
            pltpu.VMEM((_CHUNK_W,), jnp.float32),
            pltpu.VMEM((_WPW,), jnp.float32),
            pltpu.VMEM((_LANES,), jnp.float32),
            pltpu.SemaphoreType.DMA,
            pltpu.SemaphoreType.DMA,
        ],
    )(_tec_body)
    return k(ga_lin, wvec)


_BTC = _BATCH - _BSC
_RB = 8192

_G = np.zeros((_L, _N_PRED), dtype=np.float32)
for _j, (_c, _s) in enumerate(zip(_COLS, _SIGNS)):
    _G[_j, _c] = _s


def _tc_body(w_ref, x_ref, g_ref, out_ref):
    x = x_ref[...]
    g = g_ref[...]
    sel = jax.lax.dot_general(
        g, x, (((1,), (1,)), ((), ())),
        preferred_element_type=jnp.float32)
    m = jnp.max(sel, axis=0, keepdims=True)
    e = jnp.exp(sel - m)
    tot = jnp.sum(e, axis=0, keepdims=True)
    w = jnp.minimum(jnp.maximum(w_ref[0, 0], _MIN_W), _MAX_W)
    rid = lax.broadcasted_iota(jnp.int32, (_L, 1), 0)
    neg = jnp.zeros((), jnp.bool_)
    for j, s in enumerate(_SIGNS):
        if s < 0:
            neg = neg | (rid == j)
    sgn = jnp.where(neg, -1.0, 1.0).astype(jnp.float32)
    out_ref[...] = e * (w / tot) * sgn


def _delta_tc(ground_atoms, w11):
    return pl.pallas_call(
        _tc_body,
        grid=(_BTC // _RB,),
        in_specs=[
            pl.BlockSpec(memory_space=pltpu.SMEM),
            pl.BlockSpec((_RB, _N_PRED), lambda i: (i + _BSC // _RB, 0)),
            pl.BlockSpec((_L, _N_PRED), lambda i: (0, 0)),
        ],
        out_specs=pl.BlockSpec((_L, _RB), lambda i: (0, i)),
        out_shape=jax.ShapeDtypeStruct((_L, _BTC), jnp.float32),
        compiler_params=pltpu.CompilerParams(
            dimension_semantics=("arbitrary",)),
    )(w11, ground_atoms, jnp.asarray(_G))


@jax.jit
def _delta(ground_atoms, wvec, w11):
    ga_lin = (
        ground_atoms.reshape(_BATCH // 8, 8, _N_PRED // 128, 128)
        .transpose(0, 2, 1, 3)
        .reshape(-1)
    )
    flat_sc = _delta_sc(ga_lin, wvec)
    delta_tc_t = _delta_tc(ground_atoms, w11)
    delta_sc = (
        flat_sc.reshape(_BSC // 128, _L, 128)
        .transpose(0, 2, 1)
        .reshape(_BSC, _L)
    )
    return jnp.concatenate([delta_sc, delta_tc_t.T], axis=0)


def kernel(ground_atoms, clause_weight):
    wvec = jnp.broadcast_to(jnp.reshape(clause_weight, (1,)), (_LANES,))
    w11 = jnp.reshape(clause_weight, (1, 1))
    delta = _delta(ground_atoms, wvec, w11)
    return (delta, jnp.asarray(_IDX_CONST))
